# Initial kernel scaffold; baseline (speedup 1.0000x reference)
import jax
import jax.numpy as jnp
from jax.experimental import pallas as pl


def kernel(dist, nidx, score):
    raise NotImplementedError("write your pallas kernel here")


# speedup vs baseline: 130.6376x; 130.6376x over previous
import functools

import jax
import jax.numpy as jnp
from jax import lax
from jax.experimental import pallas as pl
from jax.experimental.pallas import tpu as pltpu
from jax.experimental.pallas import tpu_sc as plsc

V = 100000
K = 64
NC = 2
NS = 16
NW = NC * NS
L = 16

ROWS_PER_W = 3136
LAST_ROWS = V - (NW - 1) * ROWS_PER_W
CHUNK = 32
CHUNKS_FULL = ROWS_PER_W // CHUNK
CHUNKS_LAST = LAST_ROWS // CHUNK


@functools.lru_cache(maxsize=1)
def _make_sc_diff():
    mesh = plsc.VectorSubcoreMesh(core_axis_name="c", subcore_axis_name="s",
                                  num_cores=NC, num_subcores=NS)

    @functools.partial(
        pl.kernel,
        mesh=mesh,
        out_type=jax.ShapeDtypeStruct((V,), jnp.float32),
        scratch_types=[
            pltpu.VMEM((V,), jnp.float32),
            pltpu.VMEM((CHUNK * K,), jnp.int32),
            pltpu.VMEM((ROWS_PER_W,), jnp.float32),
        ],
        compiler_params=pltpu.CompilerParams(needs_layout_passes=False),
    )
    def sc_diff(score_hbm, nidx_hbm, diff_hbm, score_v, nidx_v, diff_v):
        wid = lax.axis_index("s") * NC + lax.axis_index("c")
        wbase = wid * ROWS_PER_W
        nchunks = jnp.where(wid == NW - 1, CHUNKS_LAST, CHUNKS_FULL)

        pltpu.sync_copy(score_hbm, score_v)

        lanes = lax.iota(jnp.int32, 16)

        def chunk_body(c, carry):
            word_base = (wbase + c * CHUNK) * K
            pltpu.sync_copy(nidx_hbm.at[pl.ds(word_base, CHUNK * K)], nidx_v)
            for g in range(CHUNK // L):
                rowoff = (lanes + g * L) * K
                srow = score_v[pl.ds(wbase + c * CHUNK + g * L, L)]
                idx = plsc.load_gather(nidx_v, [rowoff + 1])
                acc = plsc.load_gather(score_v, [idx])
                for k in range(2, K):
                    idx = plsc.load_gather(nidx_v, [rowoff + k])
                    s = plsc.load_gather(score_v, [idx])
                    acc = jnp.maximum(acc, s)
                diff_v[pl.ds(c * CHUNK + g * L, L)] = srow - acc
            return carry

        lax.fori_loop(0, nchunks, chunk_body, 0)

        @pl.when(wid == NW - 1)
        def _():
            pltpu.sync_copy(diff_v.at[pl.ds(0, LAST_ROWS)],
                            diff_hbm.at[pl.ds(wbase, LAST_ROWS)])

        @pl.when(wid != NW - 1)
        def _():
            pltpu.sync_copy(diff_v, diff_hbm.at[pl.ds(wbase, ROWS_PER_W)])

    return sc_diff


R2 = V * K // 128
TB = 1000


def _tc_body(diff_ref, dist_ref, nidx_ref, do_ref, no_ref):
    d = diff_ref[...]
    dfull = jnp.concatenate(
        [jnp.broadcast_to(d[:, 0:1], (TB, 64)),
         jnp.broadcast_to(d[:, 1:2], (TB, 64))], axis=1)
    mask = dfull < 0.0
    dist_blk = dist_ref[...]
    nidx_blk = nidx_ref[...]
    do_ref[...] = jnp.where(mask, 0.0, dist_blk)
    col = lax.broadcasted_iota(jnp.int32, (TB, 128), 1) & 63
    noneigh = jnp.where(col == 0, nidx_blk, -1)
    no_ref[...] = jnp.where(mask, noneigh, nidx_blk)


_tc_select = pl.pallas_call(
    _tc_body,
    grid=(R2 // TB,),
    in_specs=[pl.BlockSpec((TB, 2), lambda i: (i, 0)),
              pl.BlockSpec((TB, 128), lambda i: (i, 0)),
              pl.BlockSpec((TB, 128), lambda i: (i, 0))],
    out_specs=[pl.BlockSpec((TB, 128), lambda i: (i, 0)),
               pl.BlockSpec((TB, 128), lambda i: (i, 0))],
    out_shape=[jax.ShapeDtypeStruct((R2, 128), jnp.float32),
               jax.ShapeDtypeStruct((R2, 128), jnp.int32)],
)


def kernel(dist, nidx, score):
    diff = _make_sc_diff()(score.reshape(-1), nidx.reshape(-1))
    do2, no2 = _tc_select(diff.reshape(R2, 2),
                          dist.reshape(R2, 128),
                          nidx.reshape(R2, 128))
    return do2.reshape(V, K), no2.reshape(V, K)
